# baseline (speedup 1.0000x reference)
import functools

import jax
import jax.numpy as jnp
from jax import lax
from jax.experimental import pallas as pl
from jax.experimental.pallas import tpu as pltpu
from jax.experimental.pallas import tpu_sc as plsc

_VOCAB = 30522
_EMB = 128
_HID = 256
_TYPE_VOCAB = 2
_B = 128
_S = 512
_NTOK = _B * _S
_LN_EPS = 1e-12

_NC = 2
_NS = 16
_NW = _NC * _NS
_ROWS_PER_W = _NTOK // _NW
_GATHER = 128
_CH_ROWS = 256
_GPC = _CH_ROWS // _GATHER
_NCHUNK = _ROWS_PER_W // _CH_ROWS


def _make_sc_body(rows_per_w, nchunk, row0):
    def body(table_hbm, idx_hbm, out_hbm, idx_v, buf0, buf1,
             gsem0, gsem1, wsem0, wsem1):
        wid = lax.axis_index("s") * _NC + lax.axis_index("c")
        base = wid * rows_per_w
        pltpu.sync_copy(idx_hbm.at[pl.ds(row0 + base, rows_per_w)], idx_v)

        bufs = (buf0, buf1)
        gsems = (gsem0, gsem1)
        wsems = (wsem0, wsem1)

        def start_gathers(c):
            b, s = bufs[c % 2], gsems[c % 2]
            return [
                pltpu.async_copy(
                    table_hbm.at[idx_v.at[pl.ds(c * _CH_ROWS + g * _GATHER, _GATHER)]],
                    b.at[pl.ds(g * _GATHER, _GATHER)], s)
                for g in range(_GPC)
            ]

        gdesc = {0: start_gathers(0)}
        wdesc = {}
        for c in range(nchunk):
            if c + 1 < nchunk:
                if c - 1 >= 0:
                    wdesc[c - 1].wait()
                gdesc[c + 1] = start_gathers(c + 1)
            for d in gdesc[c]:
                d.wait()
            wdesc[c] = pltpu.async_copy(
                bufs[c % 2], out_hbm.at[pl.ds(base + c * _CH_ROWS, _CH_ROWS)],
                wsems[c % 2])
        for c in range(max(0, nchunk - 2), nchunk):
            wdesc[c].wait()
    return body


@functools.cache
def _sc_gather(nrows, row0):
    rows_per_w = nrows // _NW
    nchunk = rows_per_w // _CH_ROWS
    return pl.kernel(
        _make_sc_body(rows_per_w, nchunk, row0),
        out_type=jax.ShapeDtypeStruct((nrows, _EMB), jnp.float32),
        mesh=plsc.VectorSubcoreMesh(core_axis_name="c", subcore_axis_name="s"),
        scratch_types=[
            pltpu.VMEM((rows_per_w,), jnp.int32),
            pltpu.VMEM((_CH_ROWS, _EMB), jnp.float32),
            pltpu.VMEM((_CH_ROWS, _EMB), jnp.float32),
            pltpu.SemaphoreType.DMA,
            pltpu.SemaphoreType.DMA,
            pltpu.SemaphoreType.DMA,
            pltpu.SemaphoreType.DMA,
        ],
    )


_SEQ_BLK = 8
_NCHB = 4
_B_CH = _B // _NCHB


def _tc_compute(e_ref, pos_ref, type_ref, gamma_ref, beta_ref, w_ref, b_ref,
                out_ref):
    t = type_ref[...]
    e = e_ref[...] + pos_ref[...][None] + t[0][None, None, :]
    mu = jnp.mean(e, axis=-1, keepdims=True)
    d = e - mu
    var = jnp.mean(d * d, axis=-1, keepdims=True)
    n = d * lax.rsqrt(var + _LN_EPS)
    n = n * gamma_ref[...][None, None, :] + beta_ref[...][None, None, :]
    out = lax.dot_general(n, w_ref[...], (((2,), (0,)), ((), ())),
                          preferred_element_type=jnp.float32)
    out_ref[...] = out + b_ref[...][None, None, :]


def _tc_body_first(e_ref, pos_ref, type_ref, gamma_ref, beta_ref, w_ref, b_ref,
                   out_ref):
    _tc_compute(e_ref, pos_ref, type_ref, gamma_ref, beta_ref, w_ref, b_ref,
                out_ref)


def _tc_body_alias(prev_ref, e_ref, pos_ref, type_ref, gamma_ref, beta_ref,
                   w_ref, b_ref, out_ref):
    del prev_ref
    _tc_compute(e_ref, pos_ref, type_ref, gamma_ref, beta_ref, w_ref, b_ref,
                out_ref)


_W_SPECS = [
    pl.BlockSpec((_S, _EMB), lambda i: (0, 0)),
    pl.BlockSpec((_TYPE_VOCAB, _EMB), lambda i: (0, 0)),
    pl.BlockSpec((_EMB,), lambda i: (0,)),
    pl.BlockSpec((_EMB,), lambda i: (0,)),
    pl.BlockSpec((_EMB, _HID), lambda i: (0, 0)),
    pl.BlockSpec((_HID,), lambda i: (0,)),
]


def _tc_chunk(prev, gathered_j, j, weights):
    e3 = gathered_j.reshape(_B_CH, _S, _EMB)
    nblk = _B_CH // _SEQ_BLK
    jbase = j * nblk
    e_spec = pl.BlockSpec((_SEQ_BLK, _S, _EMB), lambda i: (i, 0, 0))
    out_spec = pl.BlockSpec((_SEQ_BLK, _S, _HID),
                            lambda i, jbase=jbase: (jbase + i, 0, 0))
    out_shape = jax.ShapeDtypeStruct((_B, _S, _HID), jnp.float32)
    if prev is None:
        return pl.pallas_call(
            _tc_body_first,
            grid=(nblk,),
            in_specs=[e_spec] + _W_SPECS,
            out_specs=out_spec,
            out_shape=out_shape,
        )(e3, *weights)
    return pl.pallas_call(
        _tc_body_alias,
        grid=(nblk,),
        in_specs=[pl.BlockSpec(memory_space=pl.ANY), e_spec] + _W_SPECS,
        out_specs=out_spec,
        out_shape=out_shape,
        input_output_aliases={0: 0},
    )(prev, e3, *weights)


def kernel(xs, word_emb, pos_emb, type_emb, ln_gamma, ln_beta, proj_W, proj_b):
    idx = xs.reshape(_NTOK)
    weights = (pos_emb, type_emb, ln_gamma, ln_beta, proj_W, proj_b)
    nrows = _NTOK // _NCHB
    gathered = [
        _sc_gather(nrows, j * nrows)(word_emb, idx)
        for j in range(_NCHB)
    ]
    out = None
    for j in range(_NCHB):
        out = _tc_chunk(out, gathered[j], j, weights)
    return out


# speedup vs baseline: 1.0890x; 1.0890x over previous
import functools

import jax
import jax.numpy as jnp
from jax import lax
from jax.experimental import pallas as pl
from jax.experimental.pallas import tpu as pltpu
from jax.experimental.pallas import tpu_sc as plsc

_VOCAB = 30522
_EMB = 128
_HID = 256
_TYPE_VOCAB = 2
_B = 128
_S = 512
_NTOK = _B * _S
_LN_EPS = 1e-12

_NC = 2
_NS = 16
_NW = _NC * _NS
_ROWS_PER_W = _NTOK // _NW
_GATHER = 128
_CH_ROWS = 256
_GPC = _CH_ROWS // _GATHER
_NCHUNK = _ROWS_PER_W // _CH_ROWS


def _make_sc_body(rows_per_w, nchunk, row0):
    def body(table_hbm, idx_hbm, out_hbm, idx_v, buf0, buf1,
             gsem0, gsem1, wsem0, wsem1):
        wid = lax.axis_index("s") * _NC + lax.axis_index("c")
        base = wid * rows_per_w
        pltpu.sync_copy(idx_hbm.at[pl.ds(row0 + base, rows_per_w)], idx_v)

        bufs = (buf0, buf1)
        gsems = (gsem0, gsem1)
        wsems = (wsem0, wsem1)

        def start_gathers(c):
            b, s = bufs[c % 2], gsems[c % 2]
            return [
                pltpu.async_copy(
                    table_hbm.at[idx_v.at[pl.ds(c * _CH_ROWS + g * _GATHER, _GATHER)]],
                    b.at[pl.ds(g * _GATHER, _GATHER)], s)
                for g in range(_GPC)
            ]

        gdesc = {0: start_gathers(0)}
        wdesc = {}
        for c in range(nchunk):
            if c + 1 < nchunk:
                if c - 1 >= 0:
                    wdesc[c - 1].wait()
                gdesc[c + 1] = start_gathers(c + 1)
            for d in gdesc[c]:
                d.wait()
            wdesc[c] = pltpu.async_copy(
                bufs[c % 2], out_hbm.at[pl.ds(base + c * _CH_ROWS, _CH_ROWS)],
                wsems[c % 2])
        for c in range(max(0, nchunk - 2), nchunk):
            wdesc[c].wait()
    return body


@functools.cache
def _sc_gather(nrows, row0):
    rows_per_w = nrows // _NW
    nchunk = rows_per_w // _CH_ROWS
    return pl.kernel(
        _make_sc_body(rows_per_w, nchunk, row0),
        out_type=jax.ShapeDtypeStruct((nrows, _EMB), jnp.float32),
        mesh=plsc.VectorSubcoreMesh(core_axis_name="c", subcore_axis_name="s"),
        scratch_types=[
            pltpu.VMEM((rows_per_w,), jnp.int32),
            pltpu.VMEM((_CH_ROWS, _EMB), jnp.float32),
            pltpu.VMEM((_CH_ROWS, _EMB), jnp.float32),
            pltpu.SemaphoreType.DMA,
            pltpu.SemaphoreType.DMA,
            pltpu.SemaphoreType.DMA,
            pltpu.SemaphoreType.DMA,
        ],
    )


_SEQ_BLK = 16
_NCHB = 1
_B_CH = _B // _NCHB


def _tc_compute(e_ref, pos_ref, type_ref, gamma_ref, beta_ref, w_ref, b_ref,
                out_ref):
    t = type_ref[...]
    e = e_ref[...] + pos_ref[...][None] + t[0][None, None, :]
    mu = jnp.mean(e, axis=-1, keepdims=True)
    d = e - mu
    var = jnp.mean(d * d, axis=-1, keepdims=True)
    n = d * lax.rsqrt(var + _LN_EPS)
    n = n * gamma_ref[...][None, None, :] + beta_ref[...][None, None, :]
    out = lax.dot_general(n, w_ref[...], (((2,), (0,)), ((), ())),
                          preferred_element_type=jnp.float32)
    out_ref[...] = out + b_ref[...][None, None, :]


def _tc_body_first(e_ref, pos_ref, type_ref, gamma_ref, beta_ref, w_ref, b_ref,
                   out_ref):
    _tc_compute(e_ref, pos_ref, type_ref, gamma_ref, beta_ref, w_ref, b_ref,
                out_ref)


def _tc_body_alias(prev_ref, e_ref, pos_ref, type_ref, gamma_ref, beta_ref,
                   w_ref, b_ref, out_ref):
    del prev_ref
    _tc_compute(e_ref, pos_ref, type_ref, gamma_ref, beta_ref, w_ref, b_ref,
                out_ref)


_W_SPECS = [
    pl.BlockSpec((_S, _EMB), lambda i: (0, 0)),
    pl.BlockSpec((_TYPE_VOCAB, _EMB), lambda i: (0, 0)),
    pl.BlockSpec((_EMB,), lambda i: (0,)),
    pl.BlockSpec((_EMB,), lambda i: (0,)),
    pl.BlockSpec((_EMB, _HID), lambda i: (0, 0)),
    pl.BlockSpec((_HID,), lambda i: (0,)),
]


def _tc_chunk(prev, gathered_j, j, weights):
    e3 = gathered_j.reshape(_B_CH, _S, _EMB)
    nblk = _B_CH // _SEQ_BLK
    jbase = j * nblk
    e_spec = pl.BlockSpec((_SEQ_BLK, _S, _EMB), lambda i: (i, 0, 0))
    out_spec = pl.BlockSpec((_SEQ_BLK, _S, _HID),
                            lambda i, jbase=jbase: (jbase + i, 0, 0))
    out_shape = jax.ShapeDtypeStruct((_B, _S, _HID), jnp.float32)
    if prev is None:
        return pl.pallas_call(
            _tc_body_first,
            grid=(nblk,),
            in_specs=[e_spec] + _W_SPECS,
            out_specs=out_spec,
            out_shape=out_shape,
        )(e3, *weights)
    return pl.pallas_call(
        _tc_body_alias,
        grid=(nblk,),
        in_specs=[pl.BlockSpec(memory_space=pl.ANY), e_spec] + _W_SPECS,
        out_specs=out_spec,
        out_shape=out_shape,
        input_output_aliases={0: 0},
    )(prev, e3, *weights)


def kernel(xs, word_emb, pos_emb, type_emb, ln_gamma, ln_beta, proj_W, proj_b):
    idx = xs.reshape(_NTOK)
    weights = (pos_emb, type_emb, ln_gamma, ln_beta, proj_W, proj_b)
    nrows = _NTOK // _NCHB
    gathered = [
        _sc_gather(nrows, j * nrows)(word_emb, idx)
        for j in range(_NCHB)
    ]
    out = None
    for j in range(_NCHB):
        out = _tc_chunk(out, gathered[j], j, weights)
    return out
